# baseline (speedup 1.0000x reference)
import functools
import jax
import jax.numpy as jnp
from jax import lax
from jax.experimental import pallas as pl
from jax.experimental.pallas import tpu as pltpu
from jax.experimental.pallas import tpu_sc as plsc


def _make_lookup(num_entities, num_relations, embed_dim, batch):
    info = plsc.get_sparse_core_info()
    nc, ns = info.num_cores, info.num_subcores
    nw = nc * ns
    b_per_w = batch // nw

    mesh = plsc.VectorSubcoreMesh(core_axis_name="c", subcore_axis_name="s")

    @functools.partial(
        pl.kernel,
        mesh=mesh,
        compiler_params=pltpu.CompilerParams(use_tc_tiling_on_sc=False),
        out_type=[
            jax.ShapeDtypeStruct((batch, embed_dim), jnp.float32),
            jax.ShapeDtypeStruct((batch, embed_dim), jnp.float32),
        ],
        scratch_types=[
            pltpu.VMEM((b_per_w,), jnp.int32),
            pltpu.VMEM((b_per_w,), jnp.int32),
            pltpu.VMEM((b_per_w, embed_dim), jnp.float32),
            pltpu.VMEM((b_per_w, embed_dim), jnp.float32),
            pltpu.SemaphoreType.DMA,
            pltpu.SemaphoreType.DMA,
        ],
    )
    def lookup(ent_hbm, rel_hbm, eid_hbm, rid_hbm, ent_out, rel_out,
               eidx_v, ridx_v, erows_v, rrows_v, esem, rsem):
        wid = lax.axis_index("s") * nc + lax.axis_index("c")
        base = wid * b_per_w
        pltpu.sync_copy(eid_hbm.at[pl.ds(base, b_per_w)], eidx_v)
        pltpu.sync_copy(rid_hbm.at[pl.ds(base, b_per_w)], ridx_v)
        ec = pltpu.async_copy(ent_hbm.at[eidx_v], erows_v, esem)
        rc = pltpu.async_copy(rel_hbm.at[ridx_v], rrows_v, rsem)
        ec.wait()
        pltpu.sync_copy(erows_v, ent_out.at[pl.ds(base, b_per_w)])
        rc.wait()
        pltpu.sync_copy(rrows_v, rel_out.at[pl.ds(base, b_per_w)])

    return lookup


def kernel(entity_embedding, relation_embedding, entity_ids, relation_ids):
    num_entities, embed_dim = entity_embedding.shape
    num_relations = relation_embedding.shape[0]
    batch = entity_ids.shape[0]
    lookup = _make_lookup(num_entities, num_relations, embed_dim, batch)
    return tuple(
        lookup(
            entity_embedding,
            relation_embedding,
            entity_ids.astype(jnp.int32),
            relation_ids.astype(jnp.int32),
        )
    )
